# baseline (speedup 1.0000x reference)
import functools
import math

import jax
import jax.numpy as jnp
from jax import lax
from jax.experimental import pallas as pl
from jax.experimental.pallas import tpu as pltpu
from jax.experimental.pallas import tpu_sc as plsc

_G0, _G1, _G2, _G3, _F = 64, 64, 24, 24, 32
_N = 262144

_NC = 2
_NS = 16
_NW = _NC * _NS
_Q = 128
_NW_Q = _N // _NW
_NB = _NW_Q // _Q

_PREP_B = 2048
_MLP_B = 4096


def _prep_body(x_ref, idx_ref, w_ref):
    pi = math.pi
    x0 = x_ref[:, 0]
    x1 = x_ref[:, 1]
    x2 = x_ref[:, 2]
    x3 = x_ref[:, 3]
    t0 = (x0 - 0.0) / (pi - 0.0) * (_G0 - 1)
    t1 = (x1 - (-pi)) / (pi - (-pi)) * (_G1 - 1)
    t2 = (x2 - 0.5 * pi) / (0.85 * pi - 0.5 * pi) * (_G2 - 1)
    t3 = (x3 - (-0.85 * pi)) / (-0.5 * pi - (-0.85 * pi)) * (_G3 - 1)
    tlx = t0.astype(jnp.int32)
    tly = t1.astype(jnp.int32)
    tlz = t2.astype(jnp.int32)
    tlw = t3.astype(jnp.int32)
    xf = t0 % 1
    yf = t1 % 1
    zf = t2 % 1
    wf = t3 % 1
    brx = jnp.minimum(tlx + 1, _G1 - 1)
    bry = jnp.minimum(tly + 1, _G0 - 1)
    brz = jnp.minimum(tlz + 1, _G2 - 1)
    brw = jnp.minimum(tlw + 1, _G3 - 1)
    ws = []
    for k in range(16):
        bx, by, bz, bw = (k >> 3) & 1, (k >> 2) & 1, (k >> 1) & 1, k & 1
        iy = bry if by else tly
        ix = brx if bx else tlx
        iz = brz if bz else tlz
        iw = brw if bw else tlw
        flat = ((iy * _G1 + ix) * _G2 + iz) * _G3 + iw
        ax = xf if bx else 1.0 - xf
        ay = yf if by else 1.0 - yf
        zsel, wsel = (bw, bz) if by else (bz, bw)
        az = zf if zsel else 1.0 - zf
        aw = wf if wsel else 1.0 - wf
        idx_ref[k, :] = flat
        ws.append(ax * ay * az * aw)
    w_ref[...] = jnp.stack(ws, axis=1)


def _prep_call(x):
    return pl.pallas_call(
        _prep_body,
        grid=(_N // _PREP_B,),
        in_specs=[pl.BlockSpec((_PREP_B, 4), lambda i: (i, 0))],
        out_specs=[
            pl.BlockSpec((16, _PREP_B), lambda i: (0, i)),
            pl.BlockSpec((_PREP_B, 16), lambda i: (i, 0)),
        ],
        out_shape=[
            jax.ShapeDtypeStruct((16, _N), jnp.int32),
            jax.ShapeDtypeStruct((_N, 16), jnp.float32),
        ],
    )(x)


def _sc_body(tab_hbm, idx_hbm, w_hbm, out_hbm, idx_v, w_v, rows_v, out_v, sem):
    wid = lax.axis_index("s") * _NC + lax.axis_index("c")
    base = wid * _NW_Q

    def block(b, carry):
        qbase = base + b * _Q
        pltpu.sync_copy(idx_hbm.at[:, pl.ds(qbase, _Q)], idx_v)
        pltpu.sync_copy(w_hbm.at[pl.ds(qbase, _Q), :], w_v)
        copies = []
        for k in range(16):
            copies.append(
                pltpu.async_copy(tab_hbm.at[idx_v.at[k]], rows_v.at[k], sem))
        for c in copies:
            c.wait()

        def q_step(q, c2):
            wq = w_v[q, 0:16]
            acc_lo = jnp.zeros((16,), jnp.float32)
            acc_hi = jnp.zeros((16,), jnp.float32)
            for k in range(16):
                wk = wq[k]
                acc_lo = acc_lo + wk * rows_v[k, q, 0:16]
                acc_hi = acc_hi + wk * rows_v[k, q, 16:32]
            out_v[q, 0:16] = acc_lo
            out_v[q, 16:32] = acc_hi
            return c2

        lax.fori_loop(0, _Q, q_step, 0, unroll=False)
        pltpu.sync_copy(out_v, out_hbm.at[pl.ds(qbase, _Q)])
        return carry

    lax.fori_loop(0, _NB, block, 0, unroll=False)


def _sc_call(tab, idx16, w16):
    mesh = plsc.VectorSubcoreMesh(core_axis_name="c", subcore_axis_name="s")
    f = functools.partial(
        pl.kernel,
        out_type=jax.ShapeDtypeStruct((_N, _F), jnp.float32),
        mesh=mesh,
        scratch_types=[
            pltpu.VMEM((16, _Q), jnp.int32),
            pltpu.VMEM((_Q, 16), jnp.float32),
            pltpu.VMEM((16, _Q, _F), jnp.float32),
            pltpu.VMEM((_Q, _F), jnp.float32),
            pltpu.SemaphoreType.DMA,
        ],
        compiler_params=pltpu.CompilerParams(use_tc_tiling_on_sc=False),
    )(_sc_body)
    return f(tab, idx16, w16)


def _mlp_body(v_ref, w1_ref, b1_ref, w2_ref, b2_ref, o_ref):
    h = jnp.dot(v_ref[...], w1_ref[...], preferred_element_type=jnp.float32)
    h = h + b1_ref[...]
    h = jnp.where(h >= 0, h, 0.01 * h)
    o = jnp.dot(h, w2_ref[...], preferred_element_type=jnp.float32)
    o = o + b2_ref[...]
    o_ref[...] = jax.nn.sigmoid(o) * 255.0


def _mlp_call(v, W1, b1, W2, b2):
    return pl.pallas_call(
        _mlp_body,
        grid=(_N // _MLP_B,),
        in_specs=[
            pl.BlockSpec((_MLP_B, _F), lambda i: (i, 0)),
            pl.BlockSpec((_F, 4 * _F), lambda i: (0, 0)),
            pl.BlockSpec((1, 4 * _F), lambda i: (0, 0)),
            pl.BlockSpec((4 * _F, 3), lambda i: (0, 0)),
            pl.BlockSpec((1, 3), lambda i: (0, 0)),
        ],
        out_specs=pl.BlockSpec((_MLP_B, 3), lambda i: (i, 0)),
        out_shape=jax.ShapeDtypeStruct((_N, 3), jnp.float32),
    )(v, W1, b1, W2, b2)


def kernel(x, grid, W1, b1, W2, b2):
    tab = grid.reshape(-1, _F)
    idx16, w16 = _prep_call(x)
    v = _sc_call(tab, idx16, w16)
    return _mlp_call(v, W1, b1.reshape(1, -1), W2, b2.reshape(1, -1))


# speedup vs baseline: 2.6069x; 2.6069x over previous
import functools
import math

import jax
import jax.numpy as jnp
from jax import lax
from jax.experimental import pallas as pl
from jax.experimental.pallas import tpu as pltpu
from jax.experimental.pallas import tpu_sc as plsc

_G0, _G1, _G2, _G3, _F = 64, 64, 24, 24, 32
_N = 262144

_NC = 2
_NS = 16
_NW = _NC * _NS
_Q = 128
_NW_Q = _N // _NW
_NB = _NW_Q // _Q

_PREP_B = 2048
_MLP_B = 4096


def _prep_body(x_ref, idx_ref, w_ref):
    pi = math.pi
    x0 = x_ref[0]
    x1 = x_ref[1]
    x2 = x_ref[2]
    x3 = x_ref[3]
    t0 = (x0 - 0.0) / (pi - 0.0) * (_G0 - 1)
    t1 = (x1 - (-pi)) / (pi - (-pi)) * (_G1 - 1)
    t2 = (x2 - 0.5 * pi) / (0.85 * pi - 0.5 * pi) * (_G2 - 1)
    t3 = (x3 - (-0.85 * pi)) / (-0.5 * pi - (-0.85 * pi)) * (_G3 - 1)
    tlx = t0.astype(jnp.int32)
    tly = t1.astype(jnp.int32)
    tlz = t2.astype(jnp.int32)
    tlw = t3.astype(jnp.int32)
    xf = t0 % 1
    yf = t1 % 1
    zf = t2 % 1
    wf = t3 % 1
    brx = jnp.minimum(tlx + 1, _G1 - 1)
    bry = jnp.minimum(tly + 1, _G0 - 1)
    brz = jnp.minimum(tlz + 1, _G2 - 1)
    brw = jnp.minimum(tlw + 1, _G3 - 1)
    ws = []
    for k in range(16):
        bx, by, bz, bw = (k >> 3) & 1, (k >> 2) & 1, (k >> 1) & 1, k & 1
        iy = bry if by else tly
        ix = brx if bx else tlx
        iz = brz if bz else tlz
        iw = brw if bw else tlw
        flat = ((iy * _G1 + ix) * _G2 + iz) * _G3 + iw
        ax = xf if bx else 1.0 - xf
        ay = yf if by else 1.0 - yf
        zsel, wsel = (bw, bz) if by else (bz, bw)
        az = zf if zsel else 1.0 - zf
        aw = wf if wsel else 1.0 - wf
        idx_ref[k] = flat
        w_ref[k] = ax * ay * az * aw


_PREP_ROWS = 64


def _prep_call(x):
    nrows = _N // 128
    xt = x.T.reshape(4, nrows, 128)
    idx3, w3 = pl.pallas_call(
        _prep_body,
        grid=(nrows // _PREP_ROWS,),
        in_specs=[pl.BlockSpec((4, _PREP_ROWS, 128), lambda i: (0, i, 0))],
        out_specs=[
            pl.BlockSpec((16, _PREP_ROWS, 128), lambda i: (0, i, 0)),
            pl.BlockSpec((16, _PREP_ROWS, 128), lambda i: (0, i, 0)),
        ],
        out_shape=[
            jax.ShapeDtypeStruct((16, nrows, 128), jnp.int32),
            jax.ShapeDtypeStruct((16, nrows, 128), jnp.float32),
        ],
    )(xt)
    return idx3.reshape(16, _N), w3.reshape(16, _N)


def _sc_body(tab_hbm, idx_hbm, w_hbm, out_hbm, idx_v, w_v, rows_v, out_v, sem):
    wid = lax.axis_index("s") * _NC + lax.axis_index("c")
    base = wid * _NW_Q

    def block(b, carry):
        qbase = base + b * _Q
        pltpu.sync_copy(idx_hbm.at[:, pl.ds(qbase, _Q)], idx_v)
        pltpu.sync_copy(w_hbm.at[:, pl.ds(qbase, _Q)], w_v)
        copies = []
        for k in range(16):
            copies.append(
                pltpu.async_copy(tab_hbm.at[idx_v.at[k]], rows_v.at[k], sem))
        for c in copies:
            c.wait()

        def grp_step(g, c2):
            qb = g * 16
            wvecs = [w_v[k, pl.ds(qb, 16)] for k in range(16)]
            for j in range(16):
                q = qb + j
                acc_lo = jnp.zeros((16,), jnp.float32)
                acc_hi = jnp.zeros((16,), jnp.float32)
                for k in range(16):
                    wk = wvecs[k][j]
                    acc_lo = acc_lo + wk * rows_v[k, q, 0:16]
                    acc_hi = acc_hi + wk * rows_v[k, q, 16:32]
                out_v[q, 0:16] = acc_lo
                out_v[q, 16:32] = acc_hi
            return c2

        lax.fori_loop(0, _Q // 16, grp_step, 0, unroll=False)
        pltpu.sync_copy(out_v, out_hbm.at[pl.ds(qbase, _Q)])
        return carry

    lax.fori_loop(0, _NB, block, 0, unroll=False)


def _sc_call(tab, idx16, w16):
    mesh = plsc.VectorSubcoreMesh(core_axis_name="c", subcore_axis_name="s")
    f = functools.partial(
        pl.kernel,
        out_type=jax.ShapeDtypeStruct((_N, _F), jnp.float32),
        mesh=mesh,
        scratch_types=[
            pltpu.VMEM((16, _Q), jnp.int32),
            pltpu.VMEM((16, _Q), jnp.float32),
            pltpu.VMEM((16, _Q, _F), jnp.float32),
            pltpu.VMEM((_Q, _F), jnp.float32),
            pltpu.SemaphoreType.DMA,
        ],
        compiler_params=pltpu.CompilerParams(use_tc_tiling_on_sc=False),
    )(_sc_body)
    return f(tab, idx16, w16)


def _mlp_body(v_ref, w1_ref, b1_ref, w2_ref, b2_ref, o_ref):
    h = jnp.dot(v_ref[...], w1_ref[...], preferred_element_type=jnp.float32)
    h = h + b1_ref[...]
    h = jnp.where(h >= 0, h, 0.01 * h)
    o = jnp.dot(h, w2_ref[...], preferred_element_type=jnp.float32)
    o = o + b2_ref[...]
    o_ref[...] = jax.nn.sigmoid(o) * 255.0


def _mlp_call(v, W1, b1, W2, b2):
    return pl.pallas_call(
        _mlp_body,
        grid=(_N // _MLP_B,),
        in_specs=[
            pl.BlockSpec((_MLP_B, _F), lambda i: (i, 0)),
            pl.BlockSpec((_F, 4 * _F), lambda i: (0, 0)),
            pl.BlockSpec((1, 4 * _F), lambda i: (0, 0)),
            pl.BlockSpec((4 * _F, 3), lambda i: (0, 0)),
            pl.BlockSpec((1, 3), lambda i: (0, 0)),
        ],
        out_specs=pl.BlockSpec((_MLP_B, 3), lambda i: (i, 0)),
        out_shape=jax.ShapeDtypeStruct((_N, 3), jnp.float32),
    )(v, W1, b1, W2, b2)


def kernel(x, grid, W1, b1, W2, b2):
    tab = grid.reshape(-1, _F)
    idx16, w16 = _prep_call(x)
    v = _sc_call(tab, idx16, w16)
    return _mlp_call(v, W1, b1.reshape(1, -1), W2, b2.reshape(1, -1))
